# baseline (speedup 1.0000x reference)
import functools

import jax
import jax.numpy as jnp
from jax import lax
from jax.experimental import pallas as pl
from jax.experimental.pallas import tpu as pltpu
from jax.experimental.pallas import tpu_sc as plsc

B = 16384
D0, D1 = 64, 32
D = D0 + D1

NC = 2
NS = 16
NW = NC * NS
BW = B // NW
L = 16
CH = 16
NCH = BW // CH
SLOT = CH * 8
OUTW = 384
ROWS_PER_W = B * D // OUTW // NW


def _sc_pair_body(ids, t0, t1, out, idx, g0, g1, cat, sem_a, sem_b):
    wid = lax.axis_index("s") * NC + lax.axis_index("c")
    base = wid * BW
    pltpu.sync_copy(ids.at[pl.ds(base, BW)], idx)

    def issue(c, soff, sem):
        vec = idx[pl.ds(c * CH, L)]
        blk = (vec >> 3) << 3
        blk1 = (vec >> 4) << 3
        for j in range(L):
            b = pl.multiple_of(blk[j], 8)
            b1 = pl.multiple_of(blk1[j], 8)
            pltpu.async_copy(t0.at[pl.ds(b, 8)],
                             g0.at[pl.ds(soff + j * 8, 8)], sem)
            pltpu.async_copy(t1.at[pl.ds(b1, 8)],
                             g1.at[pl.ds(soff + j * 8, 8)], sem)

    def drain(sem):
        pltpu.make_async_copy(t0.at[pl.ds(0, SLOT)],
                              g0.at[pl.ds(0, SLOT)], sem).wait()
        pltpu.make_async_copy(t1.at[pl.ds(0, SLOT)],
                              g1.at[pl.ds(0, SLOT)], sem).wait()

    def assemble(c, soff):
        vec = idx[pl.ds(c * CH, L)]
        svec = vec & 7
        s1vec = (vec >> 1) & 7
        h1vec = (vec & 1) << 5
        for j in range(L):
            s = svec[j]
            r0 = soff + j * 8 + s
            r1 = soff + j * 8 + s1vec[j]
            h1 = pl.multiple_of(h1vec[j], 16)
            row = (c * CH + j) // 4
            colbase = D * (j % 4)
            for k in range(D0 // L):
                cat[row, pl.ds(colbase + k * L, L)] = \
                    g0[r0, pl.ds(k * L, L)]
            for k in range(D1 // L):
                cat[row, pl.ds(colbase + D0 + k * L, L)] = \
                    g1[r1, pl.ds(h1 + k * L, L)]

    def superstep(t, _):
        c0 = 2 * t
        issue(c0 + 1, SLOT, sem_b)
        drain(sem_a)
        assemble(c0, 0)

        @pl.when(t < NCH // 2 - 1)
        def _():
            issue(c0 + 2, 0, sem_a)

        drain(sem_b)
        assemble(c0 + 1, SLOT)
        return ()

    issue(0, 0, sem_a)
    lax.fori_loop(0, NCH // 2, superstep, ())
    pltpu.sync_copy(cat, out.at[pl.ds(wid * ROWS_PER_W, ROWS_PER_W)])


def _make_pair_kernel():
    mesh = plsc.VectorSubcoreMesh(core_axis_name="c", subcore_axis_name="s")
    return functools.partial(
        pl.kernel,
        out_type=jax.ShapeDtypeStruct((B * D // OUTW, OUTW), jnp.float32),
        mesh=mesh,
        scratch_types=[
            pltpu.VMEM((BW,), jnp.int32),
            pltpu.VMEM((2 * SLOT, D0), jnp.float32),
            pltpu.VMEM((2 * SLOT, D0), jnp.float32),
            pltpu.VMEM((ROWS_PER_W, OUTW), jnp.float32),
            pltpu.SemaphoreType.DMA,
            pltpu.SemaphoreType.DMA,
        ],
    )(_sc_pair_body)


def kernel(item_ids, user_ids, item_table_0, user_table_0, item_table_1, user_table_1):
    run = _make_pair_kernel()
    o_u = run(user_ids, user_table_0,
              user_table_1.reshape(user_table_1.shape[0] // 2, 2 * D1))
    o_i = run(item_ids, item_table_0,
              item_table_1.reshape(item_table_1.shape[0] // 2, 2 * D1))
    return o_i.reshape(B, D), o_u.reshape(B, D)


# speedup vs baseline: 1.1960x; 1.1960x over previous
import functools

import jax
import jax.numpy as jnp
from jax import lax
from jax.experimental import pallas as pl
from jax.experimental.pallas import tpu as pltpu
from jax.experimental.pallas import tpu_sc as plsc

B = 16384
D0, D1 = 64, 32
D = D0 + D1

NC = 2
NS = 16
NW = NC * NS
BW = B // NW
L = 16
CH = 16
NCH = BW // CH
SLOT = CH * 8
OUTW = 384
ROWS_PER_W = B * D // OUTW // NW


def _sc_pair_body(ids, t0, t1, out, idx, g0, g1, cat, sem_a, sem_b):
    wid = lax.axis_index("s") * NC + lax.axis_index("c")
    base = wid * BW
    pltpu.sync_copy(ids.at[pl.ds(base, BW)], idx)

    def issue(c, soff, sem):
        vec = idx[pl.ds(c * CH, L)]
        blk = (vec >> 3) << 3
        for j in range(L):
            b = pl.multiple_of(blk[j], 8)
            pltpu.async_copy(t0.at[pl.ds(b, 8)],
                             g0.at[pl.ds(soff + j * 8, 8)], sem)
            pltpu.async_copy(t1.at[pl.ds(b, 8)],
                             g1.at[pl.ds(soff + j * 8, 8)], sem)

    def drain(sem):
        pltpu.make_async_copy(t0.at[pl.ds(0, SLOT)],
                              g0.at[pl.ds(0, SLOT)], sem).wait()
        pltpu.make_async_copy(t1.at[pl.ds(0, SLOT)],
                              g1.at[pl.ds(0, SLOT)], sem).wait()

    def assemble(c, soff):
        svec = idx[pl.ds(c * CH, L)] & 7
        for j in range(L):
            s = svec[j]
            r0 = soff + j * 8 + s
            row = (c * CH + j) // 4
            colbase = D * (j % 4)
            for k in range(D0 // L):
                cat[row, pl.ds(colbase + k * L, L)] = \
                    g0[r0, pl.ds(k * L, L)]
            for k in range(D1 // L):
                cat[row, pl.ds(colbase + D0 + k * L, L)] = \
                    g1[r0, pl.ds(k * L, L)]

    def superstep(t, _):
        c0 = 2 * t
        issue(c0 + 1, SLOT, sem_b)
        drain(sem_a)
        assemble(c0, 0)

        @pl.when(t < NCH // 2 - 1)
        def _():
            issue(c0 + 2, 0, sem_a)

        drain(sem_b)
        assemble(c0 + 1, SLOT)
        return ()

    issue(0, 0, sem_a)
    lax.fori_loop(0, NCH // 2, superstep, ())
    pltpu.sync_copy(cat, out.at[pl.ds(wid * ROWS_PER_W, ROWS_PER_W)])


def _make_pair_kernel():
    mesh = plsc.VectorSubcoreMesh(core_axis_name="c", subcore_axis_name="s")
    return functools.partial(
        pl.kernel,
        out_type=jax.ShapeDtypeStruct((B * D // OUTW, OUTW), jnp.float32),
        mesh=mesh,
        scratch_types=[
            pltpu.VMEM((BW,), jnp.int32),
            pltpu.VMEM((2 * SLOT, D0), jnp.float32),
            pltpu.VMEM((2 * SLOT, D1), jnp.float32),
            pltpu.VMEM((ROWS_PER_W, OUTW), jnp.float32),
            pltpu.SemaphoreType.DMA,
            pltpu.SemaphoreType.DMA,
        ],
    )(_sc_pair_body)


def kernel(item_ids, user_ids, item_table_0, user_table_0, item_table_1, user_table_1):
    run = _make_pair_kernel()
    o_u = run(user_ids, user_table_0, user_table_1)
    o_i = run(item_ids, item_table_0, item_table_1)
    return o_i.reshape(B, D), o_u.reshape(B, D)
